# baseline (speedup 1.0000x reference)
import functools

import jax
import jax.numpy as jnp
from jax import lax
from jax.experimental import pallas as pl
from jax.experimental.pallas import tpu as pltpu
from jax.experimental.pallas import tpu_sc as plsc

HID = 768
L = 16
NH = HID // L
NC = 2
NS = 16
NW = NC * NS
B = 4
S = 2048
PPW = S // NW
C = 16
CPB = PPW // C
NCHUNK = B * CPB
TB = 8
EPS = 1e-12


def _rsqrt(xv):
    yi = lax.bitcast_convert_type(xv, jnp.int32)
    y = lax.bitcast_convert_type(
        jnp.int32(0x5F3759DF) - lax.shift_right_logical(yi, 1), jnp.float32)
    for _ in range(3):
        y = y * (1.5 - 0.5 * xv * y * y)
    return y


def _lane_sum(v):
    for sh in (8, 4, 2, 1):
        perm = lax.broadcasted_iota(jnp.int32, (L,), 0) ^ sh
        v = v + v.at[perm].get(mode="promise_in_bounds")
    return v


mesh = plsc.VectorSubcoreMesh(core_axis_name="c", subcore_axis_name="s")


@functools.partial(
    pl.kernel,
    mesh=mesh,
    out_type=jax.ShapeDtypeStruct((B * S, HID), jnp.float32),
    compiler_params=pltpu.CompilerParams(needs_layout_passes=False),
    scratch_types=[
        pltpu.VMEM((B * PPW,), jnp.int32),
        pltpu.VMEM((B * PPW,), jnp.int32),
        pltpu.VMEM((B * PPW,), jnp.float32),
        pltpu.VMEM((C, HID), jnp.float32),
        pltpu.VMEM((C, HID), jnp.float32),
        pltpu.VMEM((C, HID), jnp.float32),
        pltpu.VMEM((C, HID), jnp.float32),
        pltpu.VMEM((C, HID), jnp.float32),
        pltpu.VMEM((PPW, HID), jnp.float32),
        pltpu.VMEM((2, HID), jnp.float32),
        pltpu.VMEM((HID,), jnp.float32),
        pltpu.VMEM((HID,), jnp.float32),
        pltpu.SemaphoreType.DMA,
        pltpu.SemaphoreType.DMA,
        pltpu.SemaphoreType.DMA,
        pltpu.SemaphoreType.DMA,
    ],
)
def _emb_kernel(ids_h, tts_h, we_h, pe_h, te_h, g_h, bt_h, out_h,
                idsw_v, ttw_v, ttfw_v, gb0, gb1, ebuf_v, ob0, ob1,
                pe_v, te_v, g_v, b_v, sg0, sg1, so0, so1):
    wid = lax.axis_index("s") * NC + lax.axis_index("c")
    pbase = wid * PPW
    gbs = (gb0, gb1)
    obs = (ob0, ob1)
    sgs = (sg0, sg1)
    sos = (so0, so1)

    def tbase_of(k):
        b = k // CPB
        half = k % CPB
        return b * S + pbase + half * C, half * C

    pe_cp = pltpu.make_async_copy(pe_h.at[pl.ds(pbase, PPW)], pe_v, sg1)
    pe_cp.start()
    id_cps = []
    for b in range(B):
        bs = pl.ds(b * S + pbase, PPW)
        c1 = pltpu.make_async_copy(ids_h.at[bs],
                                   idsw_v.at[pl.ds(b * PPW, PPW)], so0)
        c2 = pltpu.make_async_copy(tts_h.at[bs],
                                   ttw_v.at[pl.ds(b * PPW, PPW)], so1)
        c1.start()
        c2.start()
        id_cps += [c1, c2]
    pltpu.sync_copy(te_h, te_v)
    pltpu.sync_copy(g_h, g_v)
    pltpu.sync_copy(bt_h, b_v)

    for h in range(NH):
        hs = pl.ds(h * L, L)
        te_v[1, hs] = te_v[1, hs] - te_v[0, hs]

    for cp in id_cps:
        cp.wait()

    pltpu.async_copy(we_h.at[idsw_v.at[pl.ds(0, C)]], gb0, sg0)

    @plsc.parallel_loop(0, B * PPW // L, unroll=2)
    def ttconv(q):
        qs = pl.ds(q * L, L)
        ttfw_v[qs] = ttw_v[qs].astype(jnp.float32)

    pe_cp.wait()

    @plsc.parallel_loop(0, PPW, unroll=2)
    def fold_body(t):
        for h in range(NH):
            hs = pl.ds(h * L, L)
            pe_v[t, hs] = pe_v[t, hs] + te_v[0, hs]

    inv_h = jnp.float32(1.0 / HID)

    def pair_body(kk, _):
        for par in range(2):
            k = kk * 2 + par
            tbase, poff = tbase_of(k)
            nxt = 1 - par

            @pl.when(k < NCHUNK - 1)
            def _():
                pltpu.async_copy(we_h.at[idsw_v.at[pl.ds((k + 1) * C, C)]],
                                 gbs[nxt], sgs[nxt])

            pltpu.make_async_copy(we_h.at[idsw_v.at[pl.ds(k * C, C)]],
                                  gbs[par], sgs[par]).wait()

            fgroup = ttfw_v[pl.ds(k * C, L)]
            rows_v = gbs[par]
            out_v = obs[par]

            @pl.when(k >= 2)
            def _():
                ptbase, _ = tbase_of(k - 2)
                pltpu.make_async_copy(obs[par],
                                      out_h.at[pl.ds(ptbase, C)],
                                      sos[par]).wait()

            @plsc.parallel_loop(0, C // TB, unroll=2)
            def block_body(tb):
                t0 = tb * TB
                lane0 = t0
                fv = [
                    fgroup.at[jnp.full((L,), lane0 + j, jnp.int32)].get(
                        mode="promise_in_bounds")
                    for j in range(TB)
                ]
                zero = jnp.zeros((L,), jnp.float32)

                @plsc.parallel_loop(0, NH, carry=tuple([zero] * (2 * TB)),
                                    unroll=2)
                def moments(h, acc):
                    hs = pl.ds(h * L, L)
                    d = te_v[1, hs]
                    out = []
                    for j in range(TB):
                        e = (rows_v[t0 + j, hs] + pe_v[poff + t0 + j, hs]
                             + fv[j] * d)
                        ebuf_v[t0 + j, hs] = e
                        out.append(acc[2 * j] + e)
                        out.append(acc[2 * j + 1] + e * e)
                    return tuple(out)

                aa = []
                bb = []
                for j in range(TB):
                    mean = _lane_sum(moments[2 * j]) * inv_h
                    var = (_lane_sum(moments[2 * j + 1]) * inv_h
                           - mean * mean)
                    rstd = _rsqrt(var + jnp.float32(EPS))
                    aa.append(rstd)
                    bb.append(-mean * rstd)

                @plsc.parallel_loop(0, NH, unroll=2)
                def normalize(h):
                    hs = pl.ds(h * L, L)
                    g = g_v[hs]
                    bt = b_v[hs]
                    for j in range(TB):
                        e = ebuf_v[t0 + j, hs]
                        out_v[t0 + j, hs] = (e * aa[j] + bb[j]) * g + bt

            pltpu.async_copy(obs[par], out_h.at[pl.ds(tbase, C)], sos[par])
        return 0

    lax.fori_loop(0, NCHUNK // 2, pair_body, 0)

    for k in (NCHUNK - 2, NCHUNK - 1):
        par = k % 2
        tbase, _ = tbase_of(k)
        pltpu.make_async_copy(obs[par], out_h.at[pl.ds(tbase, C)],
                              sos[par]).wait()


def kernel(input_ids, token_type_ids, word_emb, pos_emb, type_emb, gamma, beta):
    ids = input_ids.reshape(-1).astype(jnp.int32)
    tts = token_type_ids.reshape(-1).astype(jnp.int32)
    out = _emb_kernel(ids, tts, word_emb, pos_emb, type_emb, gamma, beta)
    return out.reshape(B, S, HID)


# speedup vs baseline: 1.2979x; 1.2979x over previous
import functools

import jax
import jax.numpy as jnp
from jax import lax
from jax.experimental import pallas as pl
from jax.experimental.pallas import tpu as pltpu
from jax.experimental.pallas import tpu_sc as plsc

HID = 768
L = 16
NH = HID // L
NC = 2
NS = 16
NW = NC * NS
B = 4
S = 2048
PPW = S // NW
C = 16
CPB = PPW // C
NCHUNK = B * CPB
TB = 8
EPS = 1e-12


def _rsqrt(xv):
    yi = lax.bitcast_convert_type(xv, jnp.int32)
    y = lax.bitcast_convert_type(
        jnp.int32(0x5F3759DF) - lax.shift_right_logical(yi, 1), jnp.float32)
    for _ in range(3):
        y = y * (1.5 - 0.5 * xv * y * y)
    return y


def _lane_sum(v):
    for sh in (8, 4, 2, 1):
        perm = lax.broadcasted_iota(jnp.int32, (L,), 0) ^ sh
        v = v + v.at[perm].get(mode="promise_in_bounds")
    return v


mesh = plsc.VectorSubcoreMesh(core_axis_name="c", subcore_axis_name="s")


@functools.partial(
    pl.kernel,
    mesh=mesh,
    out_type=jax.ShapeDtypeStruct((B * S, HID), jnp.float32),
    compiler_params=pltpu.CompilerParams(needs_layout_passes=False),
    scratch_types=[
        pltpu.VMEM((B * PPW,), jnp.int32),
        pltpu.VMEM((B * PPW,), jnp.int32),
        pltpu.VMEM((B * PPW,), jnp.float32),
        pltpu.VMEM((C, HID), jnp.float32),
        pltpu.VMEM((C, HID), jnp.float32),
        pltpu.VMEM((C, HID), jnp.float32),
        pltpu.VMEM((C, HID), jnp.float32),
        pltpu.VMEM((C, HID), jnp.float32),
        pltpu.VMEM((PPW, HID), jnp.float32),
        pltpu.VMEM((2, HID), jnp.float32),
        pltpu.VMEM((HID,), jnp.float32),
        pltpu.VMEM((HID,), jnp.float32),
        pltpu.SemaphoreType.DMA,
        pltpu.SemaphoreType.DMA,
        pltpu.SemaphoreType.DMA,
        pltpu.SemaphoreType.DMA,
    ],
)
def _emb_kernel(ids_h, tts_h, we_h, pe_h, te_h, g_h, bt_h, out_h,
                idsw_v, ttw_v, ttfw_v, gb0, gb1, ebuf_v, ob0, ob1,
                pe_v, te_v, g_v, b_v, sg0, sg1, so0, so1):
    wid = lax.axis_index("s") * NC + lax.axis_index("c")
    pbase = wid * PPW
    gbs = (gb0, gb1)
    obs = (ob0, ob1)
    sgs = (sg0, sg1)
    sos = (so0, so1)

    def tbase_of(k):
        b = k // CPB
        half = k % CPB
        return b * S + pbase + half * C, half * C

    pe_cp = pltpu.make_async_copy(pe_h.at[pl.ds(pbase, PPW)], pe_v, sg1)
    pe_cp.start()
    id_cps = []
    for b in range(B):
        bs = pl.ds(b * S + pbase, PPW)
        c1 = pltpu.make_async_copy(ids_h.at[bs],
                                   idsw_v.at[pl.ds(b * PPW, PPW)], so0)
        c2 = pltpu.make_async_copy(tts_h.at[bs],
                                   ttw_v.at[pl.ds(b * PPW, PPW)], so1)
        c1.start()
        c2.start()
        id_cps += [c1, c2]
    pltpu.sync_copy(te_h, te_v)
    pltpu.sync_copy(g_h, g_v)
    pltpu.sync_copy(bt_h, b_v)

    for h in range(NH):
        hs = pl.ds(h * L, L)
        te_v[1, hs] = te_v[1, hs] - te_v[0, hs]

    for cp in id_cps:
        cp.wait()

    pltpu.async_copy(we_h.at[idsw_v.at[pl.ds(0, C)]], gb0, sg0)

    @plsc.parallel_loop(0, B * PPW // L, unroll=2)
    def ttconv(q):
        qs = pl.ds(q * L, L)
        ttfw_v[qs] = ttw_v[qs].astype(jnp.float32)

    pe_cp.wait()

    @plsc.parallel_loop(0, PPW, unroll=2)
    def fold_body(t):
        for h in range(NH):
            hs = pl.ds(h * L, L)
            pe_v[t, hs] = pe_v[t, hs] + te_v[0, hs]

    inv_h = jnp.float32(1.0 / HID)

    def pair_body(kk, _):
        for par in range(2):
            k = kk * 2 + par
            tbase, poff = tbase_of(k)
            nxt = 1 - par

            @pl.when(k < NCHUNK - 1)
            def _():
                pltpu.async_copy(we_h.at[idsw_v.at[pl.ds((k + 1) * C, C)]],
                                 gbs[nxt], sgs[nxt])

            pltpu.make_async_copy(we_h.at[idsw_v.at[pl.ds(k * C, C)]],
                                  gbs[par], sgs[par]).wait()

            fgroup = ttfw_v[pl.ds(k * C, L)]
            rows_v = gbs[par]
            out_v = obs[par]

            def block_body(tb, _):
                t0 = tb * TB
                lane0 = t0
                fv = [
                    fgroup.at[jnp.full((L,), lane0 + j, jnp.int32)].get(
                        mode="promise_in_bounds")
                    for j in range(TB)
                ]
                zero = jnp.zeros((L,), jnp.float32)

                @plsc.parallel_loop(0, NH, carry=tuple([zero] * (2 * TB)),
                                    unroll=2)
                def moments(h, acc):
                    hs = pl.ds(h * L, L)
                    d = te_v[1, hs]
                    out = []
                    for j in range(TB):
                        e = (rows_v[t0 + j, hs] + pe_v[poff + t0 + j, hs]
                             + fv[j] * d)
                        ebuf_v[t0 + j, hs] = e
                        out.append(acc[2 * j] + e)
                        out.append(acc[2 * j + 1] + e * e)
                    return tuple(out)

                aa = []
                bb = []
                for j in range(TB):
                    mean = _lane_sum(moments[2 * j]) * inv_h
                    var = (_lane_sum(moments[2 * j + 1]) * inv_h
                           - mean * mean)
                    rstd = _rsqrt(var + jnp.float32(EPS))
                    aa.append(rstd)
                    bb.append(-mean * rstd)

                @plsc.parallel_loop(0, NH, unroll=2)
                def normalize(h):
                    hs = pl.ds(h * L, L)
                    g = g_v[hs]
                    bt = b_v[hs]
                    for j in range(TB):
                        e = ebuf_v[t0 + j, hs]
                        out_v[t0 + j, hs] = (e * aa[j] + bb[j]) * g + bt
                return 0

            @pl.when(k >= 2)
            def _():
                ptbase, _ = tbase_of(k - 2)
                pltpu.make_async_copy(obs[par],
                                      out_h.at[pl.ds(ptbase, C)],
                                      sos[par]).wait()

            lax.fori_loop(0, C // TB, block_body, 0)
            pltpu.async_copy(obs[par], out_h.at[pl.ds(tbase, C)], sos[par])
        return 0

    lax.fori_loop(0, NCHUNK // 2, pair_body, 0)

    for k in (NCHUNK - 2, NCHUNK - 1):
        par = k % 2
        tbase, _ = tbase_of(k)
        pltpu.make_async_copy(obs[par], out_h.at[pl.ds(tbase, C)],
                              sos[par]).wait()


def kernel(input_ids, token_type_ids, word_emb, pos_emb, type_emb, gamma, beta):
    ids = input_ids.reshape(-1).astype(jnp.int32)
    tts = token_type_ids.reshape(-1).astype(jnp.int32)
    out = _emb_kernel(ids, tts, word_emb, pos_emb, type_emb, gamma, beta)
    return out.reshape(B, S, HID)
